# baseline (speedup 1.0000x reference)
import functools

import jax
import jax.numpy as jnp
from jax import lax
from jax.experimental import pallas as pl
from jax.experimental.pallas import tpu as pltpu
from jax.experimental.pallas import tpu_sc as plsc

B = 16384
D = 256
K = 1024
COMMIT = 0.25
TEMP = 0.5
BLK = 1024
NC = 2
NS = 16
CHUNK = 2 * B // (NC * NS)


def _vq_kernel(scR_ref, scA_ref, emb_ref, projw_ref, projb_ref,
               qR_ref, qA_ref, lossR_ref, lossA_ref, idxR_ref, idxA_ref,
               cb_ref, cbTs_ref, cbsqs_ref,
               accRR_ref, accAA_ref, accAX_ref, accRX_ref):
    i = pl.program_id(0)
    nsteps = pl.num_programs(0)
    inv_t = 1.0 / max(TEMP, 0.001)

    @pl.when(i == 0)
    def _init():
        cb = jax.lax.dot_general(
            emb_ref[...], projw_ref[...],
            (((1,), (1,)), ((), ())),
            preferred_element_type=jnp.float32) + projb_ref[...]
        cb_ref[...] = cb
        cbT = cb.T
        cbsqs_ref[...] = jnp.sum(cbT * cbT, axis=0, keepdims=True) * inv_t
        cbTs_ref[...] = cbT * (2.0 * inv_t)
        accRR_ref[...] = jnp.zeros_like(accRR_ref)
        accAA_ref[...] = jnp.zeros_like(accAA_ref)
        accAX_ref[...] = jnp.zeros_like(accAX_ref)
        accRX_ref[...] = jnp.zeros_like(accRX_ref)

    cb = cb_ref[...]
    cbTs = cbTs_ref[...]
    cbsqs = cbsqs_ref[...]

    iota = jax.lax.broadcasted_iota(
        jnp.int32, (BLK, K), 1).astype(jnp.float32)

    def process(x, q_ref, idx_ref):
        ts = cbsqs - jnp.dot(x, cbTs, preferred_element_type=jnp.float32)
        tmin = jnp.min(ts, axis=1, keepdims=True)
        e = jnp.exp(tmin - ts)
        se = jnp.sum(e, axis=1, keepdims=True)
        q_ref[...] = jnp.dot(e, cb, preferred_element_type=jnp.float32) * (1.0 / se)
        mask = ts == tmin
        idx = jnp.min(jnp.where(mask, iota, float(K)), axis=1, keepdims=True)
        idx_ref[...] = idx.astype(jnp.int32)
        return ts, tmin, mask

    def ssum(v):
        return jnp.sum(v).reshape(1, 1)

    xR = scR_ref[...]
    xA = scA_ref[...]
    tsR, tminR, maskR = process(xR, qR_ref, idxR_ref)
    tsA, tminA, maskA = process(xA, qA_ref, idxA_ref)

    sxR = ssum(xR * xR)
    sxA = ssum(xA * xA)
    accRR_ref[...] += TEMP * ssum(tminR) + sxR
    accAA_ref[...] += TEMP * ssum(tminA) + sxA
    accAX_ref[...] += TEMP * ssum(jnp.where(maskA, tsR, 0.0)) + sxR
    accRX_ref[...] += TEMP * ssum(jnp.where(maskR, tsA, 0.0)) + sxA

    @pl.when(i == nsteps - 1)
    def _finalize():
        scale = 1.0 / (B * D)
        mseRR = accRR_ref[...] * scale
        mseAA = accAA_ref[...] * scale
        mseAX = accAX_ref[...] * scale
        mseRX = accRX_ref[...] * scale
        lossR_ref[...] = 2.0 * COMMIT * mseRR
        fwd = mseAA + mseRR + 0.5 * mseAX + 0.5 * mseRX
        lossA_ref[...] = 2.0 * COMMIT * mseAA + COMMIT * fwd


def _sc_bincount(idx_all):
    mesh = plsc.VectorSubcoreMesh(core_axis_name="c", subcore_axis_name="s")

    @functools.partial(
        pl.kernel, mesh=mesh,
        out_type=jax.ShapeDtypeStruct((NC * NS, K), jnp.float32),
        scratch_types=[
            pltpu.VMEM((CHUNK,), jnp.int32),
            pltpu.VMEM((K,), jnp.float32),
        ],
        compiler_params=pltpu.CompilerParams(needs_layout_passes=False))
    def k(idx_hbm, out_hbm, idx_v, counts_v):
        c = lax.axis_index("c")
        s = lax.axis_index("s")
        w = c * NS + s
        pltpu.sync_copy(idx_hbm.at[pl.ds(w * CHUNK, CHUNK)], idx_v)
        zeros = jnp.zeros((16,), jnp.float32)

        def zbody(j, carry):
            counts_v[pl.ds(j * 16, 16)] = zeros
            return carry
        lax.fori_loop(0, K // 16, zbody, 0)

        ones = jnp.ones((16,), jnp.float32)

        def body(j, carry):
            iv = idx_v[pl.ds(j * 16, 16)]
            plsc.addupdate_scatter(counts_v, [iv], ones)
            return carry
        lax.fori_loop(0, CHUNK // 16, body, 0)
        pltpu.sync_copy(counts_v, out_hbm.at[w])

    return k(idx_all)


def _perp_kernel(p_ref, perpR_ref, perpA_ref):
    cR = jnp.sum(p_ref[0:NS, :], axis=0, keepdims=True) * (1.0 / B)
    cA = jnp.sum(p_ref[NS:2 * NS, :], axis=0, keepdims=True) * (1.0 / B)
    perpR_ref[...] = jnp.exp(-jnp.sum(cR * jnp.log(cR + 1e-10))).reshape(1, 1)
    perpA_ref[...] = jnp.exp(-jnp.sum(cA * jnp.log(cA + 1e-10))).reshape(1, 1)


@jax.jit
def _run(scR, scA, emb_weight, proj_w, proj_b_row):
    nsteps = B // BLK
    grid = (nsteps,)
    out_shapes = (
        jax.ShapeDtypeStruct((B, D), jnp.float32),
        jax.ShapeDtypeStruct((B, D), jnp.float32),
        jax.ShapeDtypeStruct((1, 1), jnp.float32),
        jax.ShapeDtypeStruct((1, 1), jnp.float32),
        jax.ShapeDtypeStruct((B, 1), jnp.int32),
        jax.ShapeDtypeStruct((B, 1), jnp.int32),
    )
    scalar_spec = pl.BlockSpec((1, 1), lambda i: (0, 0))
    out_specs = (
        pl.BlockSpec((BLK, D), lambda i: (i, 0)),
        pl.BlockSpec((BLK, D), lambda i: (i, 0)),
        scalar_spec, scalar_spec,
        pl.BlockSpec((BLK, 1), lambda i: (i, 0)),
        pl.BlockSpec((BLK, 1), lambda i: (i, 0)),
    )
    in_specs = [
        pl.BlockSpec((BLK, D), lambda i: (i, 0)),
        pl.BlockSpec((BLK, D), lambda i: (i, 0)),
        pl.BlockSpec((K, D), lambda i: (0, 0)),
        pl.BlockSpec((D, D), lambda i: (0, 0)),
        pl.BlockSpec((1, D), lambda i: (0, 0)),
    ]
    scratch = [
        pltpu.VMEM((K, D), jnp.float32),
        pltpu.VMEM((D, K), jnp.float32),
        pltpu.VMEM((1, K), jnp.float32),
        pltpu.VMEM((1, 1), jnp.float32),
        pltpu.VMEM((1, 1), jnp.float32),
        pltpu.VMEM((1, 1), jnp.float32),
        pltpu.VMEM((1, 1), jnp.float32),
    ]
    qR, qA, lossR, lossA, idxR, idxA = pl.pallas_call(
        _vq_kernel,
        grid=grid,
        in_specs=in_specs,
        out_specs=out_specs,
        out_shape=out_shapes,
        scratch_shapes=scratch,
        compiler_params=pltpu.CompilerParams(
            dimension_semantics=("arbitrary",),
        ),
    )(scR, scA, emb_weight, proj_w, proj_b_row)

    idx_all = jnp.concatenate(
        [idxR.reshape(B), idxA.reshape(B)], axis=0)
    partials = _sc_bincount(idx_all)

    perpR, perpA = pl.pallas_call(
        _perp_kernel,
        out_shape=(jax.ShapeDtypeStruct((1, 1), jnp.float32),
                   jax.ShapeDtypeStruct((1, 1), jnp.float32)),
    )(partials)
    return qR, qA, lossR, lossA, perpR, perpA


def kernel(scRNA_semantic, scATAC_semantic, flag, emb_weight, proj_w, proj_b):
    qR, qA, lossR, lossA, perpR, perpA = _run(
        scRNA_semantic, scATAC_semantic, emb_weight, proj_w,
        proj_b.reshape(1, D))
    return (qR, qA, lossR[0, 0], lossA[0, 0], perpR[0, 0], perpA[0, 0])


# speedup vs baseline: 1.3159x; 1.3159x over previous
import jax
import jax.numpy as jnp
from jax.experimental import pallas as pl
from jax.experimental.pallas import tpu as pltpu

B = 16384
D = 256
K = 1024
COMMIT = 0.25
TEMP = 0.5
BLK = 1024
LOG2E = 1.4426950408889634


def _vq_kernel(scR_ref, scA_ref, emb_ref, projw_ref, projb_ref,
               qR_ref, qA_ref, lossR_ref, lossA_ref, perpR_ref, perpA_ref,
               cb_ref, cbTs_ref, cbsqs_ref,
               accRR_ref, accAA_ref, accAX_ref, accRX_ref,
               countsR_ref, countsA_ref):
    i = pl.program_id(0)
    nsteps = pl.num_programs(0)
    inv_t = 1.0 / max(TEMP, 0.001)

    @pl.when(i == 0)
    def _init():
        cb = jax.lax.dot_general(
            emb_ref[...], projw_ref[...],
            (((1,), (1,)), ((), ())),
            preferred_element_type=jnp.float32) + projb_ref[...]
        cb_ref[...] = cb
        cbT = cb.T
        cbsqs_ref[...] = jnp.sum(cbT * cbT, axis=0, keepdims=True) * inv_t
        cbTs_ref[...] = cbT * (2.0 * inv_t)
        accRR_ref[...] = jnp.zeros_like(accRR_ref)
        accAA_ref[...] = jnp.zeros_like(accAA_ref)
        accAX_ref[...] = jnp.zeros_like(accAX_ref)
        accRX_ref[...] = jnp.zeros_like(accRX_ref)
        countsR_ref[...] = jnp.zeros_like(countsR_ref)
        countsA_ref[...] = jnp.zeros_like(countsA_ref)

    cb = cb_ref[...]
    cbTs = cbTs_ref[...]
    cbsqs = cbsqs_ref[...]

    def process(x, q_ref, counts_ref):
        ts = cbsqs - jnp.dot(x, cbTs, preferred_element_type=jnp.float32)
        tmin = jnp.min(ts, axis=1, keepdims=True)
        e = jnp.exp(tmin - ts)
        se = jnp.sum(e, axis=1, keepdims=True)
        q_ref[...] = jnp.dot(e, cb, preferred_element_type=jnp.float32) * (1.0 / se)
        mask = ts == tmin
        counts_ref[...] += jnp.sum(
            jnp.where(mask, 1.0, 0.0), axis=0, keepdims=True)
        return ts, tmin, mask

    def ssum(v):
        return jnp.sum(v).reshape(1, 1)

    xR = scR_ref[...]
    xA = scA_ref[...]
    tsR, tminR, maskR = process(xR, qR_ref, countsR_ref)
    tsA, tminA, maskA = process(xA, qA_ref, countsA_ref)

    sxR = ssum(xR * xR)
    sxA = ssum(xA * xA)
    t_ln = TEMP
    accRR_ref[...] += t_ln * ssum(tminR) + sxR
    accAA_ref[...] += t_ln * ssum(tminA) + sxA
    accAX_ref[...] += t_ln * ssum(jnp.where(maskA, tsR, 0.0)) + sxR
    accRX_ref[...] += t_ln * ssum(jnp.where(maskR, tsA, 0.0)) + sxA

    @pl.when(i == nsteps - 1)
    def _finalize():
        scale = 1.0 / (B * D)
        mseRR = accRR_ref[...] * scale
        mseAA = accAA_ref[...] * scale
        mseAX = accAX_ref[...] * scale
        mseRX = accRX_ref[...] * scale
        lossR_ref[...] = 2.0 * COMMIT * mseRR
        fwd = mseAA + mseRR + 0.5 * mseAX + 0.5 * mseRX
        lossA_ref[...] = 2.0 * COMMIT * mseAA + COMMIT * fwd
        pR = countsR_ref[...] * (1.0 / B)
        pA = countsA_ref[...] * (1.0 / B)
        perpR_ref[...] = jnp.exp(-jnp.sum(pR * jnp.log(pR + 1e-10))).reshape(1, 1)
        perpA_ref[...] = jnp.exp(-jnp.sum(pA * jnp.log(pA + 1e-10))).reshape(1, 1)


@jax.jit
def _run(scR, scA, emb_weight, proj_w, proj_b_row):
    nsteps = B // BLK
    grid = (nsteps,)
    out_shapes = (
        jax.ShapeDtypeStruct((B, D), jnp.float32),
        jax.ShapeDtypeStruct((B, D), jnp.float32),
        jax.ShapeDtypeStruct((1, 1), jnp.float32),
        jax.ShapeDtypeStruct((1, 1), jnp.float32),
        jax.ShapeDtypeStruct((1, 1), jnp.float32),
        jax.ShapeDtypeStruct((1, 1), jnp.float32),
    )
    scalar_spec = pl.BlockSpec((1, 1), lambda i: (0, 0))
    out_specs = (
        pl.BlockSpec((BLK, D), lambda i: (i, 0)),
        pl.BlockSpec((BLK, D), lambda i: (i, 0)),
        scalar_spec, scalar_spec, scalar_spec, scalar_spec,
    )
    in_specs = [
        pl.BlockSpec((BLK, D), lambda i: (i, 0)),
        pl.BlockSpec((BLK, D), lambda i: (i, 0)),
        pl.BlockSpec((K, D), lambda i: (0, 0)),
        pl.BlockSpec((D, D), lambda i: (0, 0)),
        pl.BlockSpec((1, D), lambda i: (0, 0)),
    ]
    scratch = [
        pltpu.VMEM((K, D), jnp.float32),
        pltpu.VMEM((D, K), jnp.float32),
        pltpu.VMEM((1, K), jnp.float32),
        pltpu.VMEM((1, 1), jnp.float32),
        pltpu.VMEM((1, 1), jnp.float32),
        pltpu.VMEM((1, 1), jnp.float32),
        pltpu.VMEM((1, 1), jnp.float32),
        pltpu.VMEM((1, K), jnp.float32),
        pltpu.VMEM((1, K), jnp.float32),
    ]
    return pl.pallas_call(
        _vq_kernel,
        grid=grid,
        in_specs=in_specs,
        out_specs=out_specs,
        out_shape=out_shapes,
        scratch_shapes=scratch,
        compiler_params=pltpu.CompilerParams(
            dimension_semantics=("arbitrary",),
        ),
    )(scR, scA, emb_weight, proj_w, proj_b_row)


def kernel(scRNA_semantic, scATAC_semantic, flag, emb_weight, proj_w, proj_b):
    qR, qA, lossR, lossA, perpR, perpA = _run(
        scRNA_semantic, scATAC_semantic, emb_weight, proj_w,
        proj_b.reshape(1, D))
    return (qR, qA, lossR[0, 0], lossA[0, 0], perpR[0, 0], perpA[0, 0])


# speedup vs baseline: 1.3264x; 1.0080x over previous
import jax
import jax.numpy as jnp
from jax.experimental import pallas as pl
from jax.experimental.pallas import tpu as pltpu

B = 16384
D = 256
K = 1024
COMMIT = 0.25
TEMP = 0.5
BLK = 2048
LOG2E = 1.4426950408889634


def _vq_kernel(scR_ref, scA_ref, emb_ref, projw_ref, projb_ref,
               qR_ref, qA_ref, lossR_ref, lossA_ref, perpR_ref, perpA_ref,
               cb_ref, cbTs_ref, cbsqs_ref,
               accRR_ref, accAA_ref, accAX_ref, accRX_ref,
               countsR_ref, countsA_ref):
    i = pl.program_id(0)
    nsteps = pl.num_programs(0)
    inv_t = 1.0 / max(TEMP, 0.001)

    @pl.when(i == 0)
    def _init():
        cb = jax.lax.dot_general(
            emb_ref[...], projw_ref[...],
            (((1,), (1,)), ((), ())),
            preferred_element_type=jnp.float32) + projb_ref[...]
        cb_ref[...] = cb
        cbT = cb.T
        cbsqs_ref[...] = jnp.sum(cbT * cbT, axis=0, keepdims=True) * inv_t
        cbTs_ref[...] = cbT * (2.0 * inv_t)
        accRR_ref[...] = jnp.zeros_like(accRR_ref)
        accAA_ref[...] = jnp.zeros_like(accAA_ref)
        accAX_ref[...] = jnp.zeros_like(accAX_ref)
        accRX_ref[...] = jnp.zeros_like(accRX_ref)
        countsR_ref[...] = jnp.zeros_like(countsR_ref)
        countsA_ref[...] = jnp.zeros_like(countsA_ref)

    cb = cb_ref[...]
    cbTs = cbTs_ref[...]
    cbsqs = cbsqs_ref[...]

    def process(x, q_ref, counts_ref):
        ts = cbsqs - jnp.dot(x, cbTs, preferred_element_type=jnp.float32)
        tmin = jnp.min(ts, axis=1, keepdims=True)
        e = jnp.exp(tmin - ts)
        se = jnp.sum(e, axis=1, keepdims=True)
        q_ref[...] = jnp.dot(e, cb, preferred_element_type=jnp.float32) * (1.0 / se)
        mask = ts == tmin
        counts_ref[...] += jnp.sum(
            jnp.where(mask, 1.0, 0.0), axis=0, keepdims=True)
        return ts, tmin, mask

    def ssum(v):
        return jnp.sum(v).reshape(1, 1)

    xR = scR_ref[...]
    xA = scA_ref[...]
    tsR, tminR, maskR = process(xR, qR_ref, countsR_ref)
    tsA, tminA, maskA = process(xA, qA_ref, countsA_ref)

    sxR = ssum(xR * xR)
    sxA = ssum(xA * xA)
    t_ln = TEMP
    accRR_ref[...] += t_ln * ssum(tminR) + sxR
    accAA_ref[...] += t_ln * ssum(tminA) + sxA
    accAX_ref[...] += t_ln * ssum(jnp.where(maskA, tsR, 0.0)) + sxR
    accRX_ref[...] += t_ln * ssum(jnp.where(maskR, tsA, 0.0)) + sxA

    @pl.when(i == nsteps - 1)
    def _finalize():
        scale = 1.0 / (B * D)
        mseRR = accRR_ref[...] * scale
        mseAA = accAA_ref[...] * scale
        mseAX = accAX_ref[...] * scale
        mseRX = accRX_ref[...] * scale
        lossR_ref[...] = 2.0 * COMMIT * mseRR
        fwd = mseAA + mseRR + 0.5 * mseAX + 0.5 * mseRX
        lossA_ref[...] = 2.0 * COMMIT * mseAA + COMMIT * fwd
        pR = countsR_ref[...] * (1.0 / B)
        pA = countsA_ref[...] * (1.0 / B)
        perpR_ref[...] = jnp.exp(-jnp.sum(pR * jnp.log(pR + 1e-10))).reshape(1, 1)
        perpA_ref[...] = jnp.exp(-jnp.sum(pA * jnp.log(pA + 1e-10))).reshape(1, 1)


@jax.jit
def _run(scR, scA, emb_weight, proj_w, proj_b_row):
    nsteps = B // BLK
    grid = (nsteps,)
    out_shapes = (
        jax.ShapeDtypeStruct((B, D), jnp.float32),
        jax.ShapeDtypeStruct((B, D), jnp.float32),
        jax.ShapeDtypeStruct((1, 1), jnp.float32),
        jax.ShapeDtypeStruct((1, 1), jnp.float32),
        jax.ShapeDtypeStruct((1, 1), jnp.float32),
        jax.ShapeDtypeStruct((1, 1), jnp.float32),
    )
    scalar_spec = pl.BlockSpec((1, 1), lambda i: (0, 0))
    out_specs = (
        pl.BlockSpec((BLK, D), lambda i: (i, 0)),
        pl.BlockSpec((BLK, D), lambda i: (i, 0)),
        scalar_spec, scalar_spec, scalar_spec, scalar_spec,
    )
    in_specs = [
        pl.BlockSpec((BLK, D), lambda i: (i, 0)),
        pl.BlockSpec((BLK, D), lambda i: (i, 0)),
        pl.BlockSpec((K, D), lambda i: (0, 0)),
        pl.BlockSpec((D, D), lambda i: (0, 0)),
        pl.BlockSpec((1, D), lambda i: (0, 0)),
    ]
    scratch = [
        pltpu.VMEM((K, D), jnp.float32),
        pltpu.VMEM((D, K), jnp.float32),
        pltpu.VMEM((1, K), jnp.float32),
        pltpu.VMEM((1, 1), jnp.float32),
        pltpu.VMEM((1, 1), jnp.float32),
        pltpu.VMEM((1, 1), jnp.float32),
        pltpu.VMEM((1, 1), jnp.float32),
        pltpu.VMEM((1, K), jnp.float32),
        pltpu.VMEM((1, K), jnp.float32),
    ]
    return pl.pallas_call(
        _vq_kernel,
        grid=grid,
        in_specs=in_specs,
        out_specs=out_specs,
        out_shape=out_shapes,
        scratch_shapes=scratch,
        compiler_params=pltpu.CompilerParams(
            dimension_semantics=("arbitrary",),
        ),
    )(scR, scA, emb_weight, proj_w, proj_b_row)


def kernel(scRNA_semantic, scATAC_semantic, flag, emb_weight, proj_w, proj_b):
    qR, qA, lossR, lossA, perpR, perpA = _run(
        scRNA_semantic, scATAC_semantic, emb_weight, proj_w,
        proj_b.reshape(1, D))
    return (qR, qA, lossR[0, 0], lossA[0, 0], perpR[0, 0], perpA[0, 0])
